# baseline (speedup 1.0000x reference)
import functools

import jax
import jax.numpy as jnp
from jax.experimental import pallas as pl


def _ln(x, g, b, eps=1e-5):
    m = jnp.mean(x, axis=-1, keepdims=True)
    v = jnp.mean((x - m) ** 2, axis=-1, keepdims=True)
    return (x - m) / jnp.sqrt(v + eps) * g + b


def _col_to_rows(col, r, l):
    return jnp.transpose(col.reshape(r, l, 1), (0, 2, 1)).reshape(r, l)


def _rows_to_col(rows):
    r, l = rows.shape
    return jnp.transpose(rows.reshape(r, 1, l), (0, 2, 1)).reshape(r * l, 1)


def _score_kernel(x_ref, g_ref, b_ref, w1_ref, b1_ref, w2_ref, b2_ref,
                  score_ref):
    R, N, C = x_ref.shape
    _, L = score_ref.shape
    s = _ln(x_ref[...].reshape(R * N, C), g_ref[...], b_ref[...])
    s = jax.nn.gelu(jnp.dot(s, w1_ref[...]) + b1_ref[...])
    logits = jnp.dot(s, w2_ref[...]) + b2_ref[...]
    m = jnp.max(logits, axis=-1, keepdims=True)
    e = jnp.exp(logits - m)
    score = e[:, 0:1] / jnp.sum(e, axis=-1, keepdims=True)
    score_ref[...] = _col_to_rows(score, R, L)


def _mask_kernel(score_ref, mask_ref, *, num_keep):
    B, N = score_ref.shape
    keys = jax.lax.bitcast_convert_type(score_ref[...], jnp.int32)
    k = jnp.int32(num_keep)

    def body_val(_, c):
        lo, hi = c
        mid = (lo + hi) // 2
        ge = jnp.sum((keys >= mid).astype(jnp.int32), axis=1,
                     keepdims=True) >= k
        return jnp.where(ge, mid, lo), jnp.where(ge, hi, mid)

    lo0 = jnp.zeros((B, 1), jnp.int32)
    hi0 = jnp.full((B, 1), 0x3F800001, jnp.int32)
    v, _ = jax.lax.fori_loop(0, 31, body_val, (lo0, hi0))

    n_gt = jnp.sum((keys > v).astype(jnp.int32), axis=1, keepdims=True)
    r = k - n_gt
    tie = keys == v
    idx = jax.lax.broadcasted_iota(jnp.int32, (B, N), 1)

    def body_idx(_, c):
        lo, hi = c
        mid = (lo + hi) // 2
        cnt = jnp.sum((tie & (idx < mid)).astype(jnp.int32), axis=1,
                      keepdims=True)
        ok = cnt >= r
        return jnp.where(ok, lo, mid + 1), jnp.where(ok, mid, hi)

    t_idx, _ = jax.lax.fori_loop(
        0, 11, body_idx, (jnp.zeros((B, 1), jnp.int32),
                          jnp.full((B, 1), N, jnp.int32)))

    keep = (keys > v) | (tie & (idx < t_idx))
    mask_ref[...] = keep.astype(jnp.float32)


def _mlp_kernel(x_ref, mask_ref, ln2_g, ln2_b, mlp_w1, mlp_b1, mlp_w2,
                mlp_b2, fast_ln_g, fast_ln_b, fast_w1, fast_b1, fast_w2,
                fast_b2, out_ref):
    R, N, C = x_ref.shape
    x = x_ref[...].reshape(R * N, C)
    keep = _rows_to_col(mask_ref[...]) > 0.5

    def bdot(a, w):
        return jnp.dot(a.astype(jnp.bfloat16), w.astype(jnp.bfloat16),
                       preferred_element_type=jnp.float32)

    h = _ln(x, ln2_g[...], ln2_b[...])
    h = bdot(jax.nn.gelu(bdot(h, mlp_w1[...]) + mlp_b1[...]),
             mlp_w2[...]) + mlp_b2[...]
    h2 = _ln(x, fast_ln_g[...], fast_ln_b[...])
    h2 = bdot(jax.nn.gelu(bdot(h2, fast_w1[...]) + fast_b1[...]),
              fast_w2[...]) + fast_b2[...]
    out_ref[...] = (x + jnp.where(keep, h, h2)).reshape(R, N, C)


def _full(a):
    return pl.BlockSpec(a.shape, lambda i: (0,) * a.ndim)


def kernel(x, pred_ln_g, pred_ln_b, pred_w1, pred_b1, pred_w2, pred_b2,
           ln2_g, ln2_b, mlp_w1, mlp_b1, mlp_w2, mlp_b2,
           fast_ln_g, fast_ln_b, fast_w1, fast_b1, fast_w2, fast_b2):
    B, N, C = x.shape
    num_keep = N // 2
    R = 8

    r2 = lambda a: a.reshape(1, -1)

    pred_args = (r2(pred_ln_g), r2(pred_ln_b), pred_w1, r2(pred_b1),
                 pred_w2, r2(pred_b2))
    scores = pl.pallas_call(
        _score_kernel,
        grid=(B // R,),
        in_specs=[pl.BlockSpec((R, N, C), lambda i: (i, 0, 0))]
                 + [_full(a) for a in pred_args],
        out_specs=pl.BlockSpec((R, N), lambda i: (i, 0)),
        out_shape=jax.ShapeDtypeStruct((B, N), jnp.float32),
    )(x, *pred_args)

    mask = pl.pallas_call(
        functools.partial(_mask_kernel, num_keep=num_keep),
        in_specs=[pl.BlockSpec((B, N), lambda: (0, 0))],
        out_specs=pl.BlockSpec((B, N), lambda: (0, 0)),
        out_shape=jax.ShapeDtypeStruct((B, N), jnp.float32),
    )(scores)

    mlp_args = (r2(ln2_g), r2(ln2_b), mlp_w1, r2(mlp_b1), mlp_w2,
                r2(mlp_b2), r2(fast_ln_g), r2(fast_ln_b), fast_w1,
                r2(fast_b1), fast_w2, r2(fast_b2))
    out = pl.pallas_call(
        _mlp_kernel,
        grid=(B // R,),
        in_specs=([pl.BlockSpec((R, N, C), lambda i: (i, 0, 0)),
                   pl.BlockSpec((R, N), lambda i: (i, 0))]
                  + [_full(a) for a in mlp_args]),
        out_specs=pl.BlockSpec((R, N, C), lambda i: (i, 0, 0)),
        out_shape=jax.ShapeDtypeStruct((B, N, C), x.dtype),
    )(x, mask, *mlp_args)

    return out


# speedup vs baseline: 1.0806x; 1.0806x over previous
import functools

import jax
import jax.numpy as jnp
from jax.experimental import pallas as pl


def _col_to_rows(col, r, l):
    return jnp.transpose(col.reshape(r, l, 1), (0, 2, 1)).reshape(r, l)


def _rows_to_col(rows):
    r, l = rows.shape
    return jnp.transpose(rows.reshape(r, 1, l), (0, 2, 1)).reshape(r * l, 1)


def _key_kernel(x_ref, g_ref, b_ref, w1_ref, b1_ref, w2d_ref, b2d_ref,
                key_ref):
    R, N, C = x_ref.shape
    x = x_ref[...].reshape(R * N, C)
    m = jnp.mean(x, axis=-1, keepdims=True)
    v = jnp.mean((x - m) ** 2, axis=-1, keepdims=True)
    s = (x - m) / jnp.sqrt(v + 1e-5) * g_ref[...] + b_ref[...]
    s = jax.nn.gelu(jnp.dot(s, w1_ref[...]) + b1_ref[...])
    d = jnp.dot(s, w2d_ref[...]) + b2d_ref[...]
    bits = jax.lax.bitcast_convert_type(d, jnp.int32)
    keys = jnp.where(bits >= 0, bits,
                     jnp.int32(-1) - jnp.bitwise_xor(bits,
                                                     jnp.int32(-2**31)))
    key_ref[...] = _col_to_rows(keys, R, N)


def _mask_kernel(key_ref, mask_ref, *, num_keep):
    B, N = key_ref.shape
    keys = key_ref[...]
    k = jnp.int32(num_keep)

    def count_ge(t):
        return jnp.sum((keys >= t).astype(jnp.int32), axis=1, keepdims=True)

    ge0 = count_ge(jnp.zeros((B, 1), jnp.int32)) >= k
    lo = jnp.where(ge0, jnp.int32(0), jnp.int32(-0x7F800002))
    hi = jnp.where(ge0, jnp.int32(0x7F800001), jnp.int32(0))

    def body_val(_, c):
        lo, hi = c
        mid = lo + (hi - lo) // 2
        ge = count_ge(mid) >= k
        return jnp.where(ge, mid, lo), jnp.where(ge, hi, mid)

    v, _ = jax.lax.fori_loop(0, 32, body_val, (lo, hi))

    n_gt = jnp.sum((keys > v).astype(jnp.int32), axis=1, keepdims=True)
    r = k - n_gt
    tie = keys == v
    idx = jax.lax.broadcasted_iota(jnp.int32, (B, N), 1)

    def body_idx(_, c):
        lo, hi = c
        mid = (lo + hi) // 2
        cnt = jnp.sum((tie & (idx < mid)).astype(jnp.int32), axis=1,
                      keepdims=True)
        ok = cnt >= r
        return jnp.where(ok, lo, mid + 1), jnp.where(ok, mid, hi)

    t_idx, _ = jax.lax.fori_loop(
        0, 11, body_idx, (jnp.zeros((B, 1), jnp.int32),
                          jnp.full((B, 1), N, jnp.int32)))

    keep = (keys > v) | (tie & (idx < t_idx))
    mask_ref[...] = keep.astype(jnp.float32)


def _mlp_kernel(x_ref, mask_ref, ln2_g, ln2_b, mlp_w1, mlp_b1, mlp_w2,
                mlp_b2, fast_ln_g, fast_ln_b, fast_w1, fast_b1, fast_w2,
                fast_b2, out_ref):
    R, N, C = x_ref.shape
    x = x_ref[...].reshape(R * N, C)
    keep = _rows_to_col(mask_ref[...]) > 0.5

    m = jnp.mean(x, axis=-1, keepdims=True)
    var = jnp.mean((x - m) ** 2, axis=-1, keepdims=True)
    n = (x - m) / jnp.sqrt(var + 1e-5)

    h = n * ln2_g[...] + ln2_b[...]
    h = jnp.dot(jax.nn.gelu(jnp.dot(h, mlp_w1[...]) + mlp_b1[...]),
                mlp_w2[...]) + mlp_b2[...]
    h2 = n * fast_ln_g[...] + fast_ln_b[...]
    h2 = jnp.dot(jax.nn.gelu(jnp.dot(h2, fast_w1[...]) + fast_b1[...]),
                 fast_w2[...]) + fast_b2[...]
    out_ref[...] = (x + jnp.where(keep, h, h2)).reshape(R, N, C)


def _full(a):
    return pl.BlockSpec(a.shape, lambda i: (0,) * a.ndim)


def kernel(x, pred_ln_g, pred_ln_b, pred_w1, pred_b1, pred_w2, pred_b2,
           ln2_g, ln2_b, mlp_w1, mlp_b1, mlp_w2, mlp_b2,
           fast_ln_g, fast_ln_b, fast_w1, fast_b1, fast_w2, fast_b2):
    B, N, C = x.shape
    num_keep = N // 2
    R = 8

    r2 = lambda a: a.reshape(1, -1)
    w2d = (pred_w2[:, 0] - pred_w2[:, 1]).reshape(-1, 1)
    b2d = (pred_b2[0] - pred_b2[1]).reshape(1, 1)

    pred_args = (r2(pred_ln_g), r2(pred_ln_b), pred_w1, r2(pred_b1),
                 w2d, b2d)
    keys = pl.pallas_call(
        _key_kernel,
        grid=(B // R,),
        in_specs=[pl.BlockSpec((R, N, C), lambda i: (i, 0, 0))]
                 + [_full(a) for a in pred_args],
        out_specs=pl.BlockSpec((R, N), lambda i: (i, 0)),
        out_shape=jax.ShapeDtypeStruct((B, N), jnp.int32),
    )(x, *pred_args)

    mask = pl.pallas_call(
        functools.partial(_mask_kernel, num_keep=num_keep),
        in_specs=[pl.BlockSpec((B, N), lambda: (0, 0))],
        out_specs=pl.BlockSpec((B, N), lambda: (0, 0)),
        out_shape=jax.ShapeDtypeStruct((B, N), jnp.float32),
    )(keys)

    mlp_args = (r2(ln2_g), r2(ln2_b), mlp_w1, r2(mlp_b1), mlp_w2,
                r2(mlp_b2), r2(fast_ln_g), r2(fast_ln_b), fast_w1,
                r2(fast_b1), fast_w2, r2(fast_b2))
    out = pl.pallas_call(
        _mlp_kernel,
        grid=(B // R,),
        in_specs=([pl.BlockSpec((R, N, C), lambda i: (i, 0, 0)),
                   pl.BlockSpec((R, N), lambda i: (i, 0))]
                  + [_full(a) for a in mlp_args]),
        out_specs=pl.BlockSpec((R, N, C), lambda i: (i, 0, 0)),
        out_shape=jax.ShapeDtypeStruct((B, N, C), x.dtype),
    )(x, mask, *mlp_args)

    return out
